# baseline (speedup 1.0000x reference)
import functools

import jax
import jax.numpy as jnp
from jax import lax
from jax.experimental import pallas as pl
from jax.experimental.pallas import tpu as pltpu
from jax.experimental.pallas import tpu_sc as plsc

NN = 50000
EE = 800000
DIN = 128
HID = 64
NH = 4
DK = 16
NTY = 3
NREL = 16
NLAYER = 2

NPAD = 50176
NBLK = 98
BS = 1563
NW = 32
SLABR = 1568
MZ = 6272
EP = 802816
CAPB = 806912
CHE = 256
SENT_REC = 1563 << 20


def _pack_body(src_ref, dst_ref, et_ref, pk_ref):
    d = dst_ref[...]
    b = jnp.zeros_like(d)
    for w in range(1, NW):
        b = b + jnp.where(d >= w * BS, 1, 0)
    dl = d - b * BS
    pk_ref[...] = src_ref[...] | (et_ref[...] << 16) | (dl << 20)


def _pack_records(src2, dst2, et2):
    return pl.pallas_call(
        _pack_body,
        grid=(1,),
        in_specs=[pl.BlockSpec((EP // 128, 128), lambda i: (0, 0))] * 3,
        out_specs=pl.BlockSpec((EP // 128, 128), lambda i: (0, 0)),
        out_shape=jax.ShapeDtypeStruct((EP // 128, 128), jnp.int32),
    )(src2, dst2, et2)


def _enc_body(x_ref, nt_ref, w1_ref, b1_ref, w2_ref, b2_ref, o_ref):
    x = x_ref[...]
    nt = nt_ref[...]
    h = jnp.zeros((512, HID), jnp.float32)
    for t in range(NTY):
        ht = jax.nn.relu(x @ w1_ref[t] + b1_ref[t][None, :])
        ht = ht @ w2_ref[t] + b2_ref[t][None, :]
        h = jnp.where(nt == t, ht, h)
    o_ref[...] = h


def _encoder(xp, nt2, p):
    return pl.pallas_call(
        _enc_body,
        grid=(NBLK,),
        in_specs=[
            pl.BlockSpec((512, DIN), lambda i: (i, 0)),
            pl.BlockSpec((512, 1), lambda i: (i, 0)),
            pl.BlockSpec((NTY, DIN, DIN), lambda i: (0, 0, 0)),
            pl.BlockSpec((NTY, DIN), lambda i: (0, 0)),
            pl.BlockSpec((NTY, DIN, HID), lambda i: (0, 0, 0)),
            pl.BlockSpec((NTY, HID), lambda i: (0, 0)),
        ],
        out_specs=pl.BlockSpec((512, HID), lambda i: (i, 0)),
        out_shape=jax.ShapeDtypeStruct((NPAD, HID), jnp.float32),
    )(xp, nt2, p["enc_W1"], p["enc_b1"], p["enc_W2"], p["enc_b2"])


def _qkv_body(h_ref, nt_ref, wq_ref, bq_ref, wk_ref, bk_ref, wv_ref, bv_ref,
              q_ref, k_ref, v_ref):
    h = h_ref[...]
    nt = nt_ref[...]
    q = jnp.zeros((512, HID), jnp.float32)
    k = jnp.zeros((512, HID), jnp.float32)
    v = jnp.zeros((512, HID), jnp.float32)
    for t in range(NTY):
        m = nt == t
        q = jnp.where(m, h @ wq_ref[t] + bq_ref[t][None, :], q)
        k = jnp.where(m, h @ wk_ref[t] + bk_ref[t][None, :], k)
        v = jnp.where(m, h @ wv_ref[t] + bv_ref[t][None, :], v)
    q_ref[...] = q
    k_ref[...] = k
    v_ref[...] = v


def _qkv(h, nt2, wq, bq, wk, bk, wv, bv):
    spec = pl.BlockSpec((512, HID), lambda i: (i, 0))
    wspec = pl.BlockSpec((NTY, HID, HID), lambda i: (0, 0, 0))
    bspec = pl.BlockSpec((NTY, HID), lambda i: (0, 0))
    sh = jax.ShapeDtypeStruct((NPAD, HID), jnp.float32)
    return pl.pallas_call(
        _qkv_body,
        grid=(NBLK,),
        in_specs=[spec, pl.BlockSpec((512, 1), lambda i: (i, 0)),
                  wspec, bspec, wspec, bspec, wspec, bspec],
        out_specs=[spec, spec, spec],
        out_shape=[sh, sh, sh],
    )(h, nt2, wq, bq, wk, bk, wv, bv)


def _ln_body(agg_ref, h_ref, nt_ref, al_ref, lw_ref, lb_ref, o_ref):
    agg = agg_ref[...]
    h = h_ref[...]
    nt = nt_ref[...]
    out = jnp.zeros((512, HID), jnp.float32)
    for t in range(NTY):
        alpha = al_ref[0, t]
        y = alpha * agg + (1.0 - alpha) * h
        mu = y.mean(-1, keepdims=True)
        var = ((y - mu) ** 2).mean(-1, keepdims=True)
        y = (y - mu) / jnp.sqrt(var + 1e-5) * lw_ref[t][None, :] + lb_ref[t][None, :]
        out = jnp.where(nt == t, y, out)
    o_ref[...] = out


def _lnskip(agg, h, nt2, alphas8, lw, lb):
    spec = pl.BlockSpec((512, HID), lambda i: (i, 0))
    return pl.pallas_call(
        _ln_body,
        grid=(NBLK,),
        in_specs=[spec, spec, pl.BlockSpec((512, 1), lambda i: (i, 0)),
                  pl.BlockSpec((8, 128), lambda i: (0, 0)),
                  pl.BlockSpec((NTY, HID), lambda i: (0, 0)),
                  pl.BlockSpec((NTY, HID), lambda i: (0, 0))],
        out_specs=spec,
        out_shape=jax.ShapeDtypeStruct((NPAD, HID), jnp.float32),
    )(agg, h, nt2, alphas8, lw, lb)


def _proj_body(h_ref, w_ref, b_ref, o_ref):
    o_ref[...] = h_ref[...] @ w_ref[...] + b_ref[...]


def _final_proj(h, W, b):
    return pl.pallas_call(
        _proj_body,
        grid=(NBLK,),
        in_specs=[
            pl.BlockSpec((512, HID), lambda i: (i, 0)),
            pl.BlockSpec((HID, HID), lambda i: (0, 0)),
            pl.BlockSpec((1, HID), lambda i: (0, 0)),
        ],
        out_specs=pl.BlockSpec((512, HID), lambda i: (i, 0)),
        out_shape=jax.ShapeDtypeStruct((NPAD, HID), jnp.float32),
    )(h, W, b.reshape(1, HID))


_CP_SC = pltpu.CompilerParams(use_tc_tiling_on_sc=False, needs_layout_passes=False)


def _sc_mesh():
    return plsc.VectorSubcoreMesh(core_axis_name="c", subcore_axis_name="s")


def _zero16(ref, n):
    zv = jnp.zeros((16,), jnp.float32)
    def b(i, _):
        ref[pl.ds(i * 16, 16)] = zv
        return 0
    lax.fori_loop(0, n // 16, b, 0)


def _bin_body(dst_hbm, pk_hbm, recs_hbm, cnt_hbm,
              dstb, pkb, outb, sentb, tmpb, cntv):
    c = lax.axis_index("c")
    s = lax.axis_index("s")
    w = s * 2 + c
    lo = w * BS
    hi = lo + BS
    sent = jnp.full((16,), SENT_REC, jnp.int32)
    def fill_sent(ref, n16):
        def b(i, _):
            ref[pl.ds(i * 16, 16)] = sent
            return 0
        lax.fori_loop(0, n16, b, 0)
    fill_sent(sentb, 128)
    fill_sent(outb, 256)

    def chunk(j, carry):
        f, wpos = carry
        pltpu.sync_copy(dst_hbm.at[pl.ds(j * 8192, 8192)], dstb)
        pltpu.sync_copy(pk_hbm.at[pl.ds(j * 8192, 8192)], pkb)
        def vec(i, f):
            d = dstb[pl.ds(i * 16, 16)]
            pk = pkb[pl.ds(i * 16, 16)]
            m = (d >= lo) & (d < hi)
            mi = jnp.where(m, 1, 0)
            pos = f + plsc.cumsum(mi) - 1
            plsc.store_scatter(outb, [pos], pk, mask=m)
            return f + plsc.all_reduce_population_count(m)[0]

        def flush(carry):
            f, wpos = carry
            @pl.when(f >= 2048)
            def _():
                off = pl.multiple_of(w * CAPB + wpos, 2048)
                pltpu.sync_copy(outb.at[pl.ds(0, 2048)],
                                recs_hbm.at[pl.ds(off, 2048)])
                def shift(i, _):
                    outb[pl.ds(i * 16, 16)] = outb[pl.ds(2048 + i * 16, 16)]
                    outb[pl.ds(2048 + i * 16, 16)] = sent
                    return 0
                lax.fori_loop(0, 128, shift, 0)
            return (jnp.where(f >= 2048, f - 2048, f),
                    jnp.where(f >= 2048, wpos + 2048, wpos))

        def sub(k2, carry):
            f, wpos = carry
            f = lax.fori_loop(k2 * 128, (k2 + 1) * 128, vec, f)
            f, wpos = flush((f, wpos))
            f, wpos = flush((f, wpos))
            return (f, wpos)
        return lax.fori_loop(0, 4, sub, (f, wpos))

    f, wpos = lax.fori_loop(0, EP // 8192, chunk, (jnp.int32(0), jnp.int32(0)))
    off = pl.multiple_of(w * CAPB + wpos, 2048)
    pltpu.sync_copy(outb.at[pl.ds(0, 2048)], recs_hbm.at[pl.ds(off, 2048)])
    off2 = pl.multiple_of(w * CAPB + wpos + 2048, 2048)
    pltpu.sync_copy(sentb, recs_hbm.at[pl.ds(off2, 2048)])
    cntv[...] = jnp.full((16,), wpos + f, jnp.int32)
    pltpu.sync_copy(cntv, cnt_hbm.at[pl.ds(pl.multiple_of(w * 16, 16), 16)])


def _bin_edges(dst_flat, pk_flat):
    return pl.kernel(
        _bin_body,
        out_type=(jax.ShapeDtypeStruct((NW * CAPB,), jnp.int32),
                  jax.ShapeDtypeStruct((NW * 16,), jnp.int32)),
        mesh=_sc_mesh(),
        compiler_params=_CP_SC,
        scratch_types=[
            pltpu.VMEM((8192,), jnp.int32),
            pltpu.VMEM((8192,), jnp.int32),
            pltpu.VMEM((4096,), jnp.int32),
            pltpu.VMEM((2048,), jnp.int32),
            pltpu.VMEM((2048,), jnp.int32),
            pltpu.VMEM((16,), jnp.int32),
        ],
    )(dst_flat, pk_flat)


def _layer_body(recs_hbm, cnt_hbm, qt_hbm, kt_hbm, vt_hbm,
                wqk_hbm, wv_hbm, bias_hbm,
                agg_hbm, sc_hbm,
                recb, qidx, kidx, dlb, etb, vidxb,
                qrows, krows, krows2, sbuf, abuf,
                mb, zb, wqkb, wvb, biasb, aggv, cntv,
                semq, semk):
    c = lax.axis_index("c")
    s = lax.axis_index("s")
    w = s * 2 + c
    iota = lax.iota(jnp.int32, 16)
    e4 = lax.shift_right_logical(iota, 2)
    h4 = iota & 3

    pltpu.sync_copy(cnt_hbm.at[pl.ds(pl.multiple_of(w * 16, 16), 16)], cntv)
    cnt = cntv[...][0]
    nch = (cnt + (CHE - 1)) // CHE

    pltpu.sync_copy(wqk_hbm, wqkb)
    pltpu.sync_copy(wv_hbm, wvb)
    pltpu.sync_copy(bias_hbm, biasb)

    neg = jnp.full((16,), -1e30, jnp.float32)
    def minit(i, _):
        mb[pl.ds(i * 16, 16)] = neg
        return 0
    lax.fori_loop(0, MZ // 16, minit, 0)
    _zero16(zb, MZ)

    def unpack(j, _):
        rec = recb[pl.ds(j * 16, 16)]
        srcv = rec & 0xFFFF
        etv = lax.shift_right_logical(rec, 16) & 0xF
        dlv = lax.shift_right_logical(rec, 20) & 0x7FF
        kidx[pl.ds(j * 16, 16)] = srcv
        etb[pl.ds(j * 16, 16)] = etv
        dlb[pl.ds(j * 16, 16)] = dlv
        qidx[pl.ds(j * 16, 16)] = dlv + (w * BS)
        return 0

    def p1(ch, _):
        roff = pl.multiple_of(w * CAPB + ch * CHE, CHE)
        pltpu.sync_copy(recs_hbm.at[pl.ds(roff, CHE)], recb)
        lax.fori_loop(0, CHE // 16, unpack, 0)
        cpq = pltpu.async_copy(qt_hbm.at[qidx], qrows, semq)
        cpk = pltpu.async_copy(kt_hbm.at[kidx], krows, semk)
        cpq.wait()
        cpk.wait()

        cols0 = h4 * DK

        def grp(g, _):
            rows = g * 4 + e4
            etg = plsc.load_gather(etb, [rows])
            wbase = etg * 64 + cols0
            acc = jnp.zeros((16,), jnp.float32)
            for dk in range(DK):
                col = cols0 + dk
                qg = plsc.load_gather(qrows, [rows, col])
                kg = plsc.load_gather(krows, [rows, col])
                wg = plsc.load_gather(wqkb, [wbase + dk])
                acc = acc + qg * kg * wg
            bv = plsc.load_gather(biasb, [etg * 4 + h4])
            sv = acc + bv
            sbuf[pl.ds(g * 16, 16)] = sv
            dlq = plsc.load_gather(dlb, [rows])
            idxv = dlq * 4 + h4
            sv = sbuf[pl.ds(g * 16, 16)]
            cur = plsc.load_gather(mb, [idxv])
            plsc.store_scatter(mb, [idxv], jnp.maximum(cur, sv))
            def retry(_i, _c):
                chk = plsc.load_gather(mb, [idxv])
                need = chk < sv
                @pl.when(plsc.all_reduce_population_count(need)[0] > 0)
                def _():
                    cur2 = plsc.load_gather(mb, [idxv])
                    plsc.store_scatter(mb, [idxv], jnp.maximum(cur2, sv),
                                      mask=need)
                return 0
            lax.fori_loop(0, 3, retry, 0)
            return 0
        lax.fori_loop(0, CHE // 4, grp, 0)
        soff = pl.multiple_of(w * (CAPB * 4) + ch * (CHE * 4), CHE * 4)
        pltpu.sync_copy(sbuf, sc_hbm.at[pl.ds(soff, CHE * 4)])
        return 0
    lax.fori_loop(0, nch, p1, 0)

    def p2(ch, _):
        roff = pl.multiple_of(w * CAPB + ch * CHE, CHE)
        pltpu.sync_copy(recs_hbm.at[pl.ds(roff, CHE)], recb)
        lax.fori_loop(0, CHE // 16, unpack, 0)
        soff = pl.multiple_of(w * (CAPB * 4) + ch * (CHE * 4), CHE * 4)
        pltpu.sync_copy(sc_hbm.at[pl.ds(soff, CHE * 4)], sbuf)
        def grp(g, _):
            dlq = plsc.load_gather(dlb, [g * 4 + e4])
            idxv = dlq * 4 + h4
            sv = sbuf[pl.ds(g * 16, 16)]
            mg = plsc.load_gather(mb, [idxv])
            es = jnp.exp(sv - mg)
            plsc.addupdate_scatter(zb, [idxv], es)
            return 0
        lax.fori_loop(0, CHE // 4, grp, 0)
        return 0
    lax.fori_loop(0, nch, p2, 0)

    zv16 = jnp.zeros((16,), jnp.float32)
    for hh in range(2):
        def zagg(i, _):
            def zc(j, _2):
                aggv[i, pl.ds(j * 16, 16)] = zv16
                return 0
            lax.fori_loop(0, 2, zc, 0)
            return 0
        lax.fori_loop(0, SLABR - 4, zagg, 0)

        def p3(ch, _):
            roff = pl.multiple_of(w * CAPB + ch * CHE, CHE)
            pltpu.sync_copy(recs_hbm.at[pl.ds(roff, CHE)], recb)
            lax.fori_loop(0, CHE // 16, unpack, 0)
            def vb(j, _):
                vidxb[pl.ds(j * 16, 16)] = kidx[pl.ds(j * 16, 16)] * 2 + hh
                return 0
            lax.fori_loop(0, CHE // 16, vb, 0)
            soff = pl.multiple_of(w * (CAPB * 4) + ch * (CHE * 4), CHE * 4)
            pltpu.sync_copy(sc_hbm.at[pl.ds(soff, CHE * 4)], sbuf)
            cpv = pltpu.async_copy(vt_hbm.at[vidxb], krows2, semk)

            def grp(g, _):
                dlq = plsc.load_gather(dlb, [g * 4 + e4])
                idxv = dlq * 4 + h4
                sv = sbuf[pl.ds(g * 16, 16)]
                mg = plsc.load_gather(mb, [idxv])
                zg = plsc.load_gather(zb, [idxv])
                av = jnp.exp(sv - mg) / (zg + 1e-9)
                abuf[pl.ds(g * 16, 16)] = av
                return 0
            lax.fori_loop(0, CHE // 4, grp, 0)
            cpv.wait()

            e8 = lax.shift_right_logical(iota, 1)
            h2l = iota & 1
            colsv = h2l * DK

            def medge(g, _):
                rows8 = g * 8 + e8
                etg = plsc.load_gather(etb, [rows8])
                dlg = plsc.load_gather(dlb, [rows8])
                ag = plsc.load_gather(abuf, [rows8 * 4 + (hh * 2) + h2l])
                wbase = etg * 64 + (hh * 2 + h2l) * DK
                for dk in range(DK):
                    vg = plsc.load_gather(krows2, [rows8, colsv + dk])
                    wg = plsc.load_gather(wvb, [wbase + dk])
                    plsc.addupdate_scatter(aggv, [dlg, colsv + dk], vg * wg * ag)
                return 0
            lax.fori_loop(0, CHE // 8, medge, 0)
            return 0
        lax.fori_loop(0, nch, p3, 0)

        pltpu.sync_copy(aggv.at[pl.ds(0, BS)],
                        agg_hbm.at[hh].at[pl.ds(w * BS, BS)])


def _layer_edge(recs, cnts, qt, kt, vt, wqk, wv, bias):
    return pl.kernel(
        _layer_body,
        out_type=(jax.ShapeDtypeStruct((2, NPAD, HID // 2), jnp.float32),
                  jax.ShapeDtypeStruct((NW * CAPB * 4,), jnp.float32)),
        mesh=_sc_mesh(),
        compiler_params=_CP_SC,
        scratch_types=[
            pltpu.VMEM((CHE,), jnp.int32),
            pltpu.VMEM((CHE,), jnp.int32),
            pltpu.VMEM((CHE,), jnp.int32),
            pltpu.VMEM((CHE,), jnp.int32),
            pltpu.VMEM((CHE,), jnp.int32),
            pltpu.VMEM((CHE,), jnp.int32),
            pltpu.VMEM((CHE, HID), jnp.float32),
            pltpu.VMEM((CHE, HID), jnp.float32),
            pltpu.VMEM((CHE, HID // 2), jnp.float32),
            pltpu.VMEM((CHE * 4,), jnp.float32),
            pltpu.VMEM((CHE * 4,), jnp.float32),
            pltpu.VMEM((MZ,), jnp.float32),
            pltpu.VMEM((MZ,), jnp.float32),
            pltpu.VMEM((NREL * HID,), jnp.float32),
            pltpu.VMEM((NREL * HID,), jnp.float32),
            pltpu.VMEM((NREL * NH,), jnp.float32),
            pltpu.VMEM((SLABR - 4, HID // 2), jnp.float32),
            pltpu.VMEM((16,), jnp.int32),
            pltpu.SemaphoreType.DMA,
            pltpu.SemaphoreType.DMA,
        ],
    )(recs, cnts, qt, kt, vt, wqk, wv, bias)


def kernel(x, params, node_type, edge_index, edge_type):
    p = params
    xp = jnp.zeros((NPAD, DIN), jnp.float32).at[:NN].set(x)
    nt2 = jnp.zeros((NPAD, 1), jnp.int32).at[:NN, 0].set(node_type)

    src = edge_index[0].astype(jnp.int32)
    dst = edge_index[1].astype(jnp.int32)
    et = edge_type.astype(jnp.int32)
    pad = EP - EE
    srcp = jnp.concatenate([src, jnp.zeros((pad,), jnp.int32)])
    dstp = jnp.concatenate([dst, jnp.full((pad,), 1 << 26, jnp.int32)])
    etp = jnp.concatenate([et, jnp.zeros((pad,), jnp.int32)])

    pk2 = _pack_records(srcp.reshape(EP // 128, 128),
                        dstp.reshape(EP // 128, 128),
                        etp.reshape(EP // 128, 128))
    recs, cnts = _bin_edges(dstp, pk2.reshape(EP))

    h = _encoder(xp, nt2, p)

    alphas = jax.nn.sigmoid(p["skip"])
    out = None
    for l in range(NLAYER):
        qt, kt, vt = _qkv(h, nt2, p["Wq"][l], p["bq"][l], p["Wk"][l],
                          p["bk"][l], p["Wv"][l], p["bv"][l])
        wqk = (0.25 * p["rel_q"][l] * p["rel_k"][l]
               * p["sign_k"][l][None]).reshape(-1)
        wv = (p["rel_v"][l] * p["sign_v"][l][None]).reshape(-1)
        bias = p["rel_bias"][l].reshape(-1)
        vt2 = vt.reshape(2 * NPAD, HID // 2)
        agg2, _sc = _layer_edge(recs, cnts, qt, kt, vt2, wqk, wv, bias)
        agg = agg2.transpose(1, 0, 2).reshape(NPAD, HID)
        al8 = jnp.zeros((8, 128), jnp.float32).at[0, :NTY].set(alphas[l])
        h = _lnskip(agg, h, nt2, al8, p["ln_w"][l], p["ln_b"][l])

    out = _final_proj(h, p["Wmu"], p["bmu"])
    return out[:NN]


# speedup vs baseline: 1.1516x; 1.1516x over previous
import functools

import jax
import jax.numpy as jnp
from jax import lax
from jax.experimental import pallas as pl
from jax.experimental.pallas import tpu as pltpu
from jax.experimental.pallas import tpu_sc as plsc

NN = 50000
EE = 800000
DIN = 128
HID = 64
NH = 4
DK = 16
NTY = 3
NREL = 16
NLAYER = 2

NPAD = 50176
NBLK = 98
BS = 1563
NW = 32
SLABR = 1568
MZ = 6272
EP = 802816
CAPB = 806912
CHE = 256
SENT_REC = 1563 << 20


def _pack_body(src_ref, dst_ref, et_ref, pk_ref):
    d = dst_ref[...]
    b = jnp.zeros_like(d)
    for w in range(1, NW):
        b = b + jnp.where(d >= w * BS, 1, 0)
    dl = d - b * BS
    pk_ref[...] = src_ref[...] | (et_ref[...] << 16) | (dl << 20)


def _pack_records(src2, dst2, et2):
    return pl.pallas_call(
        _pack_body,
        grid=(1,),
        in_specs=[pl.BlockSpec((EP // 128, 128), lambda i: (0, 0))] * 3,
        out_specs=pl.BlockSpec((EP // 128, 128), lambda i: (0, 0)),
        out_shape=jax.ShapeDtypeStruct((EP // 128, 128), jnp.int32),
    )(src2, dst2, et2)


def _enc_body(x_ref, nt_ref, w1_ref, b1_ref, w2_ref, b2_ref, o_ref):
    x = x_ref[...]
    nt = nt_ref[...]
    h = jnp.zeros((512, HID), jnp.float32)
    for t in range(NTY):
        ht = jax.nn.relu(x @ w1_ref[t] + b1_ref[t][None, :])
        ht = ht @ w2_ref[t] + b2_ref[t][None, :]
        h = jnp.where(nt == t, ht, h)
    o_ref[...] = h


def _encoder(xp, nt2, p):
    return pl.pallas_call(
        _enc_body,
        grid=(NBLK,),
        in_specs=[
            pl.BlockSpec((512, DIN), lambda i: (i, 0)),
            pl.BlockSpec((512, 1), lambda i: (i, 0)),
            pl.BlockSpec((NTY, DIN, DIN), lambda i: (0, 0, 0)),
            pl.BlockSpec((NTY, DIN), lambda i: (0, 0)),
            pl.BlockSpec((NTY, DIN, HID), lambda i: (0, 0, 0)),
            pl.BlockSpec((NTY, HID), lambda i: (0, 0)),
        ],
        out_specs=pl.BlockSpec((512, HID), lambda i: (i, 0)),
        out_shape=jax.ShapeDtypeStruct((NPAD, HID), jnp.float32),
    )(xp, nt2, p["enc_W1"], p["enc_b1"], p["enc_W2"], p["enc_b2"])


def _qkv_body(h_ref, nt_ref, wq_ref, bq_ref, wk_ref, bk_ref, wv_ref, bv_ref,
              q_ref, k_ref, v_ref):
    h = h_ref[...]
    nt = nt_ref[...]
    q = jnp.zeros((512, HID), jnp.float32)
    k = jnp.zeros((512, HID), jnp.float32)
    v = jnp.zeros((512, HID), jnp.float32)
    for t in range(NTY):
        m = nt == t
        q = jnp.where(m, h @ wq_ref[t] + bq_ref[t][None, :], q)
        k = jnp.where(m, h @ wk_ref[t] + bk_ref[t][None, :], k)
        v = jnp.where(m, h @ wv_ref[t] + bv_ref[t][None, :], v)
    q_ref[...] = q
    k_ref[...] = k
    v_ref[...] = v


def _qkv(h, nt2, wq, bq, wk, bk, wv, bv):
    spec = pl.BlockSpec((512, HID), lambda i: (i, 0))
    wspec = pl.BlockSpec((NTY, HID, HID), lambda i: (0, 0, 0))
    bspec = pl.BlockSpec((NTY, HID), lambda i: (0, 0))
    sh = jax.ShapeDtypeStruct((NPAD, HID), jnp.float32)
    return pl.pallas_call(
        _qkv_body,
        grid=(NBLK,),
        in_specs=[spec, pl.BlockSpec((512, 1), lambda i: (i, 0)),
                  wspec, bspec, wspec, bspec, wspec, bspec],
        out_specs=[spec, spec, spec],
        out_shape=[sh, sh, sh],
    )(h, nt2, wq, bq, wk, bk, wv, bv)


def _ln_body(agg_ref, h_ref, nt_ref, al_ref, lw_ref, lb_ref, o_ref):
    agg = agg_ref[...]
    h = h_ref[...]
    nt = nt_ref[...]
    out = jnp.zeros((512, HID), jnp.float32)
    for t in range(NTY):
        alpha = al_ref[0, t]
        y = alpha * agg + (1.0 - alpha) * h
        mu = y.mean(-1, keepdims=True)
        var = ((y - mu) ** 2).mean(-1, keepdims=True)
        y = (y - mu) / jnp.sqrt(var + 1e-5) * lw_ref[t][None, :] + lb_ref[t][None, :]
        out = jnp.where(nt == t, y, out)
    o_ref[...] = out


def _lnskip(agg, h, nt2, alphas8, lw, lb):
    spec = pl.BlockSpec((512, HID), lambda i: (i, 0))
    return pl.pallas_call(
        _ln_body,
        grid=(NBLK,),
        in_specs=[spec, spec, pl.BlockSpec((512, 1), lambda i: (i, 0)),
                  pl.BlockSpec((8, 128), lambda i: (0, 0)),
                  pl.BlockSpec((NTY, HID), lambda i: (0, 0)),
                  pl.BlockSpec((NTY, HID), lambda i: (0, 0))],
        out_specs=spec,
        out_shape=jax.ShapeDtypeStruct((NPAD, HID), jnp.float32),
    )(agg, h, nt2, alphas8, lw, lb)


def _proj_body(h_ref, w_ref, b_ref, o_ref):
    o_ref[...] = h_ref[...] @ w_ref[...] + b_ref[...]


def _final_proj(h, W, b):
    return pl.pallas_call(
        _proj_body,
        grid=(NBLK,),
        in_specs=[
            pl.BlockSpec((512, HID), lambda i: (i, 0)),
            pl.BlockSpec((HID, HID), lambda i: (0, 0)),
            pl.BlockSpec((1, HID), lambda i: (0, 0)),
        ],
        out_specs=pl.BlockSpec((512, HID), lambda i: (i, 0)),
        out_shape=jax.ShapeDtypeStruct((NPAD, HID), jnp.float32),
    )(h, W, b.reshape(1, HID))


_CP_SC = pltpu.CompilerParams(use_tc_tiling_on_sc=False, needs_layout_passes=False)


def _sc_mesh():
    return plsc.VectorSubcoreMesh(core_axis_name="c", subcore_axis_name="s")


def _zero16(ref, n):
    zv = jnp.zeros((16,), jnp.float32)
    def b(i, _):
        ref[pl.ds(i * 16, 16)] = zv
        return 0
    lax.fori_loop(0, n // 16, b, 0)


def _bin_body(dst_hbm, pk_hbm, recs_hbm, cnt_hbm,
              dstb, pkb, outb, sentb, tmpb, cntv):
    c = lax.axis_index("c")
    s = lax.axis_index("s")
    w = s * 2 + c
    lo = w * BS
    hi = lo + BS
    sent = jnp.full((16,), SENT_REC, jnp.int32)
    def fill_sent(ref, n16):
        def b(i, _):
            ref[pl.ds(i * 16, 16)] = sent
            return 0
        lax.fori_loop(0, n16, b, 0)
    fill_sent(sentb, 128)
    fill_sent(outb, 256)

    def chunk(j, carry):
        f, wpos = carry
        pltpu.sync_copy(dst_hbm.at[pl.ds(j * 8192, 8192)], dstb)
        pltpu.sync_copy(pk_hbm.at[pl.ds(j * 8192, 8192)], pkb)
        def vec(i, f):
            d = dstb[pl.ds(i * 16, 16)]
            pk = pkb[pl.ds(i * 16, 16)]
            m = (d >= lo) & (d < hi)
            mi = jnp.where(m, 1, 0)
            pos = f + plsc.cumsum(mi) - 1
            plsc.store_scatter(outb, [pos], pk, mask=m)
            return f + plsc.all_reduce_population_count(m)[0]

        def flush(carry):
            f, wpos = carry
            @pl.when(f >= 2048)
            def _():
                off = pl.multiple_of(w * CAPB + wpos, 2048)
                pltpu.sync_copy(outb.at[pl.ds(0, 2048)],
                                recs_hbm.at[pl.ds(off, 2048)])
                def shift(i, _):
                    outb[pl.ds(i * 16, 16)] = outb[pl.ds(2048 + i * 16, 16)]
                    outb[pl.ds(2048 + i * 16, 16)] = sent
                    return 0
                lax.fori_loop(0, 128, shift, 0)
            return (jnp.where(f >= 2048, f - 2048, f),
                    jnp.where(f >= 2048, wpos + 2048, wpos))

        def sub(k2, carry):
            f, wpos = carry
            f = lax.fori_loop(k2 * 128, (k2 + 1) * 128, vec, f)
            f, wpos = flush((f, wpos))
            f, wpos = flush((f, wpos))
            return (f, wpos)
        return lax.fori_loop(0, 4, sub, (f, wpos))

    f, wpos = lax.fori_loop(0, EP // 8192, chunk, (jnp.int32(0), jnp.int32(0)))
    off = pl.multiple_of(w * CAPB + wpos, 2048)
    pltpu.sync_copy(outb.at[pl.ds(0, 2048)], recs_hbm.at[pl.ds(off, 2048)])
    off2 = pl.multiple_of(w * CAPB + wpos + 2048, 2048)
    pltpu.sync_copy(sentb, recs_hbm.at[pl.ds(off2, 2048)])
    cntv[...] = jnp.full((16,), wpos + f, jnp.int32)
    pltpu.sync_copy(cntv, cnt_hbm.at[pl.ds(pl.multiple_of(w * 16, 16), 16)])


def _bin_edges(dst_flat, pk_flat):
    return pl.kernel(
        _bin_body,
        out_type=(jax.ShapeDtypeStruct((NW * CAPB,), jnp.int32),
                  jax.ShapeDtypeStruct((NW * 16,), jnp.int32)),
        mesh=_sc_mesh(),
        compiler_params=_CP_SC,
        scratch_types=[
            pltpu.VMEM((8192,), jnp.int32),
            pltpu.VMEM((8192,), jnp.int32),
            pltpu.VMEM((4096,), jnp.int32),
            pltpu.VMEM((2048,), jnp.int32),
            pltpu.VMEM((2048,), jnp.int32),
            pltpu.VMEM((16,), jnp.int32),
        ],
    )(dst_flat, pk_flat)


def _layer_body(recs_hbm, cnt_hbm, qt_hbm, kt_hbm, vt_hbm,
                wqk_hbm, wv_hbm, bias_hbm,
                agg_hbm, sc_hbm,
                recb, qidx, kidx, dlb, etb, vidxb,
                qrows, krows, krows2, sbuf, abuf,
                mb, zb, wqkb, wvb, biasb, aggv, cntv,
                semq, semk):
    c = lax.axis_index("c")
    s = lax.axis_index("s")
    w = s * 2 + c
    iota = lax.iota(jnp.int32, 16)
    e4 = lax.shift_right_logical(iota, 2)
    h4 = iota & 3

    pltpu.sync_copy(cnt_hbm.at[pl.ds(pl.multiple_of(w * 16, 16), 16)], cntv)
    cnt = cntv[...][0]
    nch = (cnt + (CHE - 1)) // CHE

    pltpu.sync_copy(wqk_hbm, wqkb)
    pltpu.sync_copy(wv_hbm, wvb)
    pltpu.sync_copy(bias_hbm, biasb)

    neg = jnp.full((16,), -1e30, jnp.float32)
    def minit(i, _):
        mb[pl.ds(i * 16, 16)] = neg
        return 0
    lax.fori_loop(0, MZ // 16, minit, 0)
    _zero16(zb, MZ)

    def unpack(j, _):
        rec = recb[pl.ds(j * 16, 16)]
        srcv = rec & 0xFFFF
        etv = lax.shift_right_logical(rec, 16) & 0xF
        dlv = lax.shift_right_logical(rec, 20) & 0x7FF
        kidx[pl.ds(j * 16, 16)] = srcv
        etb[pl.ds(j * 16, 16)] = etv
        dlb[pl.ds(j * 16, 16)] = dlv
        qidx[pl.ds(j * 16, 16)] = dlv + (w * BS)
        return 0

    def p1(ch, _):
        roff = pl.multiple_of(w * CAPB + ch * CHE, CHE)
        pltpu.sync_copy(recs_hbm.at[pl.ds(roff, CHE)], recb)
        lax.fori_loop(0, CHE // 16, unpack, 0)
        cpq = pltpu.async_copy(qt_hbm.at[qidx], qrows, semq)
        cpk = pltpu.async_copy(kt_hbm.at[kidx], krows, semk)
        cpq.wait()
        cpk.wait()

        def edge(e, _):
            etv = plsc.load_gather(etb, [jnp.full((16,), 0, jnp.int32) + e])
            wbase = etv * 64
            sv = jnp.zeros((16,), jnp.float32)
            for h in range(NH):
                qv = qrows[e, pl.ds(h * DK, DK)]
                kv = krows[e, pl.ds(h * DK, DK)]
                wv_ = plsc.load_gather(wqkb, [wbase + (h * DK) + iota])
                sh = jnp.sum(qv * kv * wv_)
                sv = sv + jnp.where(iota == h, sh, 0.0)
            bv = plsc.load_gather(biasb, [etv * 4 + h4])
            sv = sv + bv
            plsc.store_scatter(sbuf, [e * 4 + iota], sv, mask=iota < 4)
            return 0
        lax.fori_loop(0, CHE, edge, 0)

        def grp(g, _):
            dlq = plsc.load_gather(dlb, [g * 4 + e4])
            idxv = dlq * 4 + h4
            sv = sbuf[pl.ds(g * 16, 16)]
            cur = plsc.load_gather(mb, [idxv])
            plsc.store_scatter(mb, [idxv], jnp.maximum(cur, sv))
            def retry(_i, _c):
                chk = plsc.load_gather(mb, [idxv])
                need = chk < sv
                @pl.when(plsc.all_reduce_population_count(need)[0] > 0)
                def _():
                    cur2 = plsc.load_gather(mb, [idxv])
                    plsc.store_scatter(mb, [idxv], jnp.maximum(cur2, sv),
                                      mask=need)
                return 0
            lax.fori_loop(0, 3, retry, 0)
            return 0
        lax.fori_loop(0, CHE // 4, grp, 0)
        soff = pl.multiple_of(w * (CAPB * 4) + ch * (CHE * 4), CHE * 4)
        pltpu.sync_copy(sbuf, sc_hbm.at[pl.ds(soff, CHE * 4)])
        return 0
    lax.fori_loop(0, nch, p1, 0)

    def p2(ch, _):
        roff = pl.multiple_of(w * CAPB + ch * CHE, CHE)
        pltpu.sync_copy(recs_hbm.at[pl.ds(roff, CHE)], recb)
        lax.fori_loop(0, CHE // 16, unpack, 0)
        soff = pl.multiple_of(w * (CAPB * 4) + ch * (CHE * 4), CHE * 4)
        pltpu.sync_copy(sc_hbm.at[pl.ds(soff, CHE * 4)], sbuf)
        def grp(g, _):
            dlq = plsc.load_gather(dlb, [g * 4 + e4])
            idxv = dlq * 4 + h4
            sv = sbuf[pl.ds(g * 16, 16)]
            mg = plsc.load_gather(mb, [idxv])
            es = jnp.exp(sv - mg)
            plsc.addupdate_scatter(zb, [idxv], es)
            return 0
        lax.fori_loop(0, CHE // 4, grp, 0)
        return 0
    lax.fori_loop(0, nch, p2, 0)

    zv16 = jnp.zeros((16,), jnp.float32)
    for hh in range(2):
        def zagg(i, _):
            def zc(j, _2):
                aggv[i, pl.ds(j * 16, 16)] = zv16
                return 0
            lax.fori_loop(0, 2, zc, 0)
            return 0
        lax.fori_loop(0, SLABR - 4, zagg, 0)

        def p3(ch, _):
            roff = pl.multiple_of(w * CAPB + ch * CHE, CHE)
            pltpu.sync_copy(recs_hbm.at[pl.ds(roff, CHE)], recb)
            lax.fori_loop(0, CHE // 16, unpack, 0)
            def vb(j, _):
                vidxb[pl.ds(j * 16, 16)] = kidx[pl.ds(j * 16, 16)] * 2 + hh
                return 0
            lax.fori_loop(0, CHE // 16, vb, 0)
            soff = pl.multiple_of(w * (CAPB * 4) + ch * (CHE * 4), CHE * 4)
            pltpu.sync_copy(sc_hbm.at[pl.ds(soff, CHE * 4)], sbuf)
            cpv = pltpu.async_copy(vt_hbm.at[vidxb], krows2, semk)

            def grp(g, _):
                dlq = plsc.load_gather(dlb, [g * 4 + e4])
                idxv = dlq * 4 + h4
                sv = sbuf[pl.ds(g * 16, 16)]
                mg = plsc.load_gather(mb, [idxv])
                zg = plsc.load_gather(zb, [idxv])
                av = jnp.exp(sv - mg) / (zg + 1e-9)
                abuf[pl.ds(g * 16, 16)] = av
                return 0
            lax.fori_loop(0, CHE // 4, grp, 0)
            cpv.wait()

            def medge(e, _):
                etv = plsc.load_gather(etb, [jnp.full((16,), 0, jnp.int32) + e])
                wbase = etv * 64
                dl_s = plsc.load_gather(dlb, [jnp.full((16,), 0, jnp.int32) + e])[0]
                for h2 in range(2):
                    h = hh * 2 + h2
                    vv = krows2[e, pl.ds(h2 * DK, DK)]
                    wv_ = plsc.load_gather(wvb, [wbase + (h * DK) + iota])
                    av = plsc.load_gather(
                        abuf, [jnp.full((16,), 0, jnp.int32) + (e * 4 + h)])
                    aggv[dl_s, pl.ds(h2 * DK, DK)] += vv * wv_ * av
                return 0
            lax.fori_loop(0, CHE, medge, 0)
            return 0
        lax.fori_loop(0, nch, p3, 0)

        pltpu.sync_copy(aggv.at[pl.ds(0, BS)],
                        agg_hbm.at[hh].at[pl.ds(w * BS, BS)])


def _layer_edge(recs, cnts, qt, kt, vt, wqk, wv, bias):
    return pl.kernel(
        _layer_body,
        out_type=(jax.ShapeDtypeStruct((2, NPAD, HID // 2), jnp.float32),
                  jax.ShapeDtypeStruct((NW * CAPB * 4,), jnp.float32)),
        mesh=_sc_mesh(),
        compiler_params=_CP_SC,
        scratch_types=[
            pltpu.VMEM((CHE,), jnp.int32),
            pltpu.VMEM((CHE,), jnp.int32),
            pltpu.VMEM((CHE,), jnp.int32),
            pltpu.VMEM((CHE,), jnp.int32),
            pltpu.VMEM((CHE,), jnp.int32),
            pltpu.VMEM((CHE,), jnp.int32),
            pltpu.VMEM((CHE, HID), jnp.float32),
            pltpu.VMEM((CHE, HID), jnp.float32),
            pltpu.VMEM((CHE, HID // 2), jnp.float32),
            pltpu.VMEM((CHE * 4,), jnp.float32),
            pltpu.VMEM((CHE * 4,), jnp.float32),
            pltpu.VMEM((MZ,), jnp.float32),
            pltpu.VMEM((MZ,), jnp.float32),
            pltpu.VMEM((NREL * HID,), jnp.float32),
            pltpu.VMEM((NREL * HID,), jnp.float32),
            pltpu.VMEM((NREL * NH,), jnp.float32),
            pltpu.VMEM((SLABR - 4, HID // 2), jnp.float32),
            pltpu.VMEM((16,), jnp.int32),
            pltpu.SemaphoreType.DMA,
            pltpu.SemaphoreType.DMA,
        ],
    )(recs, cnts, qt, kt, vt, wqk, wv, bias)


def kernel(x, params, node_type, edge_index, edge_type):
    p = params
    xp = jnp.zeros((NPAD, DIN), jnp.float32).at[:NN].set(x)
    nt2 = jnp.zeros((NPAD, 1), jnp.int32).at[:NN, 0].set(node_type)

    src = edge_index[0].astype(jnp.int32)
    dst = edge_index[1].astype(jnp.int32)
    et = edge_type.astype(jnp.int32)
    pad = EP - EE
    srcp = jnp.concatenate([src, jnp.zeros((pad,), jnp.int32)])
    dstp = jnp.concatenate([dst, jnp.full((pad,), 1 << 26, jnp.int32)])
    etp = jnp.concatenate([et, jnp.zeros((pad,), jnp.int32)])

    pk2 = _pack_records(srcp.reshape(EP // 128, 128),
                        dstp.reshape(EP // 128, 128),
                        etp.reshape(EP // 128, 128))
    recs, cnts = _bin_edges(dstp, pk2.reshape(EP))

    h = _encoder(xp, nt2, p)

    alphas = jax.nn.sigmoid(p["skip"])
    out = None
    for l in range(NLAYER):
        qt, kt, vt = _qkv(h, nt2, p["Wq"][l], p["bq"][l], p["Wk"][l],
                          p["bk"][l], p["Wv"][l], p["bv"][l])
        wqk = (0.25 * p["rel_q"][l] * p["rel_k"][l]
               * p["sign_k"][l][None]).reshape(-1)
        wv = (p["rel_v"][l] * p["sign_v"][l][None]).reshape(-1)
        bias = p["rel_bias"][l].reshape(-1)
        vt2 = vt.reshape(2 * NPAD, HID // 2)
        agg2, _sc = _layer_edge(recs, cnts, qt, kt, vt2, wqk, wv, bias)
        agg = agg2.transpose(1, 0, 2).reshape(NPAD, HID)
        al8 = jnp.zeros((8, 128), jnp.float32).at[0, :NTY].set(alphas[l])
        h = _lnskip(agg, h, nt2, al8, p["ln_w"][l], p["ln_b"][l])

    out = _final_proj(h, p["Wmu"], p["bmu"])
    return out[:NN]


# speedup vs baseline: 1.1721x; 1.0178x over previous
import functools

import jax
import jax.numpy as jnp
from jax import lax
from jax.experimental import pallas as pl
from jax.experimental.pallas import tpu as pltpu
from jax.experimental.pallas import tpu_sc as plsc

NN = 50000
EE = 800000
DIN = 128
HID = 64
NH = 4
DK = 16
NTY = 3
NREL = 16
NLAYER = 2

NPAD = 50176
NBLK = 98
BS = 1563
NW = 32
SLABR = 1568
MZ = 6272
EP = 802816
CAPB = 806912
CHE = 256
SENT_REC = 1563 << 20


def _pack_body(src_ref, dst_ref, et_ref, pk_ref):
    d = dst_ref[...]
    b = jnp.zeros_like(d)
    for w in range(1, NW):
        b = b + jnp.where(d >= w * BS, 1, 0)
    dl = d - b * BS
    pk_ref[...] = src_ref[...] | (et_ref[...] << 16) | (dl << 20)


def _pack_records(src2, dst2, et2):
    return pl.pallas_call(
        _pack_body,
        grid=(1,),
        in_specs=[pl.BlockSpec((EP // 128, 128), lambda i: (0, 0))] * 3,
        out_specs=pl.BlockSpec((EP // 128, 128), lambda i: (0, 0)),
        out_shape=jax.ShapeDtypeStruct((EP // 128, 128), jnp.int32),
    )(src2, dst2, et2)


def _enc_body(x_ref, nt_ref, w1_ref, b1_ref, w2_ref, b2_ref, o_ref):
    x = x_ref[...]
    nt = nt_ref[...]
    h = jnp.zeros((512, HID), jnp.float32)
    for t in range(NTY):
        ht = jax.nn.relu(x @ w1_ref[t] + b1_ref[t][None, :])
        ht = ht @ w2_ref[t] + b2_ref[t][None, :]
        h = jnp.where(nt == t, ht, h)
    o_ref[...] = h


def _encoder(xp, nt2, p):
    return pl.pallas_call(
        _enc_body,
        grid=(NBLK,),
        in_specs=[
            pl.BlockSpec((512, DIN), lambda i: (i, 0)),
            pl.BlockSpec((512, 1), lambda i: (i, 0)),
            pl.BlockSpec((NTY, DIN, DIN), lambda i: (0, 0, 0)),
            pl.BlockSpec((NTY, DIN), lambda i: (0, 0)),
            pl.BlockSpec((NTY, DIN, HID), lambda i: (0, 0, 0)),
            pl.BlockSpec((NTY, HID), lambda i: (0, 0)),
        ],
        out_specs=pl.BlockSpec((512, HID), lambda i: (i, 0)),
        out_shape=jax.ShapeDtypeStruct((NPAD, HID), jnp.float32),
    )(xp, nt2, p["enc_W1"], p["enc_b1"], p["enc_W2"], p["enc_b2"])


def _qkv_body(h_ref, nt_ref, wq_ref, bq_ref, wk_ref, bk_ref, wv_ref, bv_ref,
              q_ref, k_ref, v_ref):
    h = h_ref[...]
    nt = nt_ref[...]
    q = jnp.zeros((512, HID), jnp.float32)
    k = jnp.zeros((512, HID), jnp.float32)
    v = jnp.zeros((512, HID), jnp.float32)
    for t in range(NTY):
        m = nt == t
        q = jnp.where(m, h @ wq_ref[t] + bq_ref[t][None, :], q)
        k = jnp.where(m, h @ wk_ref[t] + bk_ref[t][None, :], k)
        v = jnp.where(m, h @ wv_ref[t] + bv_ref[t][None, :], v)
    q_ref[...] = q
    k_ref[...] = k
    v_ref[...] = v


def _qkv(h, nt2, wq, bq, wk, bk, wv, bv):
    spec = pl.BlockSpec((512, HID), lambda i: (i, 0))
    wspec = pl.BlockSpec((NTY, HID, HID), lambda i: (0, 0, 0))
    bspec = pl.BlockSpec((NTY, HID), lambda i: (0, 0))
    sh = jax.ShapeDtypeStruct((NPAD, HID), jnp.float32)
    return pl.pallas_call(
        _qkv_body,
        grid=(NBLK,),
        in_specs=[spec, pl.BlockSpec((512, 1), lambda i: (i, 0)),
                  wspec, bspec, wspec, bspec, wspec, bspec],
        out_specs=[spec, spec, spec],
        out_shape=[sh, sh, sh],
    )(h, nt2, wq, bq, wk, bk, wv, bv)


def _ln_body(agg_ref, h_ref, nt_ref, al_ref, lw_ref, lb_ref, o_ref):
    agg = agg_ref[...]
    h = h_ref[...]
    nt = nt_ref[...]
    out = jnp.zeros((512, HID), jnp.float32)
    for t in range(NTY):
        alpha = al_ref[0, t]
        y = alpha * agg + (1.0 - alpha) * h
        mu = y.mean(-1, keepdims=True)
        var = ((y - mu) ** 2).mean(-1, keepdims=True)
        y = (y - mu) / jnp.sqrt(var + 1e-5) * lw_ref[t][None, :] + lb_ref[t][None, :]
        out = jnp.where(nt == t, y, out)
    o_ref[...] = out


def _lnskip(agg, h, nt2, alphas8, lw, lb):
    spec = pl.BlockSpec((512, HID), lambda i: (i, 0))
    return pl.pallas_call(
        _ln_body,
        grid=(NBLK,),
        in_specs=[spec, spec, pl.BlockSpec((512, 1), lambda i: (i, 0)),
                  pl.BlockSpec((8, 128), lambda i: (0, 0)),
                  pl.BlockSpec((NTY, HID), lambda i: (0, 0)),
                  pl.BlockSpec((NTY, HID), lambda i: (0, 0))],
        out_specs=spec,
        out_shape=jax.ShapeDtypeStruct((NPAD, HID), jnp.float32),
    )(agg, h, nt2, alphas8, lw, lb)


def _proj_body(h_ref, w_ref, b_ref, o_ref):
    o_ref[...] = h_ref[...] @ w_ref[...] + b_ref[...]


def _final_proj(h, W, b):
    return pl.pallas_call(
        _proj_body,
        grid=(NBLK,),
        in_specs=[
            pl.BlockSpec((512, HID), lambda i: (i, 0)),
            pl.BlockSpec((HID, HID), lambda i: (0, 0)),
            pl.BlockSpec((1, HID), lambda i: (0, 0)),
        ],
        out_specs=pl.BlockSpec((512, HID), lambda i: (i, 0)),
        out_shape=jax.ShapeDtypeStruct((NPAD, HID), jnp.float32),
    )(h, W, b.reshape(1, HID))


_CP_SC = pltpu.CompilerParams(use_tc_tiling_on_sc=False, needs_layout_passes=False)


def _sc_mesh():
    return plsc.VectorSubcoreMesh(core_axis_name="c", subcore_axis_name="s")


def _zero16(ref, n):
    zv = jnp.zeros((16,), jnp.float32)
    def b(i, _):
        ref[pl.ds(i * 16, 16)] = zv
        return 0
    lax.fori_loop(0, n // 16, b, 0)


def _bin_body(dst_hbm, pk_hbm, recs_hbm, cnt_hbm,
              dstb, pkb, outb, sentb, tmpb, cntv):
    c = lax.axis_index("c")
    s = lax.axis_index("s")
    w = s * 2 + c
    lo = w * BS
    hi = lo + BS
    sent = jnp.full((16,), SENT_REC, jnp.int32)
    def fill_sent(ref, n16):
        def b(i, _):
            ref[pl.ds(i * 16, 16)] = sent
            return 0
        lax.fori_loop(0, n16, b, 0)
    fill_sent(sentb, 128)
    fill_sent(outb, 256)

    def chunk(j, carry):
        f, wpos = carry
        pltpu.sync_copy(dst_hbm.at[pl.ds(j * 8192, 8192)], dstb)
        pltpu.sync_copy(pk_hbm.at[pl.ds(j * 8192, 8192)], pkb)
        def vec(i, f):
            d = dstb[pl.ds(i * 16, 16)]
            pk = pkb[pl.ds(i * 16, 16)]
            m = (d >= lo) & (d < hi)
            mi = jnp.where(m, 1, 0)
            pos = f + plsc.cumsum(mi) - 1
            plsc.store_scatter(outb, [pos], pk, mask=m)
            return f + plsc.all_reduce_population_count(m)[0]

        def flush(carry):
            f, wpos = carry
            @pl.when(f >= 2048)
            def _():
                off = pl.multiple_of(w * CAPB + wpos, 2048)
                pltpu.sync_copy(outb.at[pl.ds(0, 2048)],
                                recs_hbm.at[pl.ds(off, 2048)])
                def shift(i, _):
                    outb[pl.ds(i * 16, 16)] = outb[pl.ds(2048 + i * 16, 16)]
                    outb[pl.ds(2048 + i * 16, 16)] = sent
                    return 0
                lax.fori_loop(0, 128, shift, 0)
            return (jnp.where(f >= 2048, f - 2048, f),
                    jnp.where(f >= 2048, wpos + 2048, wpos))

        def sub(k2, carry):
            f, wpos = carry
            f = lax.fori_loop(k2 * 128, (k2 + 1) * 128, vec, f)
            f, wpos = flush((f, wpos))
            f, wpos = flush((f, wpos))
            return (f, wpos)
        return lax.fori_loop(0, 4, sub, (f, wpos))

    f, wpos = lax.fori_loop(0, EP // 8192, chunk, (jnp.int32(0), jnp.int32(0)))
    off = pl.multiple_of(w * CAPB + wpos, 2048)
    pltpu.sync_copy(outb.at[pl.ds(0, 2048)], recs_hbm.at[pl.ds(off, 2048)])
    off2 = pl.multiple_of(w * CAPB + wpos + 2048, 2048)
    pltpu.sync_copy(sentb, recs_hbm.at[pl.ds(off2, 2048)])
    cntv[...] = jnp.full((16,), wpos + f, jnp.int32)
    pltpu.sync_copy(cntv, cnt_hbm.at[pl.ds(pl.multiple_of(w * 16, 16), 16)])


def _bin_edges(dst_flat, pk_flat):
    return pl.kernel(
        _bin_body,
        out_type=(jax.ShapeDtypeStruct((NW * CAPB,), jnp.int32),
                  jax.ShapeDtypeStruct((NW * 16,), jnp.int32)),
        mesh=_sc_mesh(),
        compiler_params=_CP_SC,
        scratch_types=[
            pltpu.VMEM((8192,), jnp.int32),
            pltpu.VMEM((8192,), jnp.int32),
            pltpu.VMEM((4096,), jnp.int32),
            pltpu.VMEM((2048,), jnp.int32),
            pltpu.VMEM((2048,), jnp.int32),
            pltpu.VMEM((16,), jnp.int32),
        ],
    )(dst_flat, pk_flat)


def _layer_body(recs_hbm, cnt_hbm, qt_hbm, kt_hbm, vt_hbm,
                wqk_hbm, wv_hbm, bias_hbm,
                agg_hbm, sc_hbm,
                recb, recb2, sbuf2, qidx, kidx, dlb, etb, vidxb,
                qrows, krows, krows2, sbuf, abuf,
                mb, zb, wqkb, wvb, biasb, aggv, cntv,
                semq, semk):
    c = lax.axis_index("c")
    s = lax.axis_index("s")
    w = s * 2 + c
    iota = lax.iota(jnp.int32, 16)
    e4 = lax.shift_right_logical(iota, 2)
    h4 = iota & 3

    pltpu.sync_copy(cnt_hbm.at[pl.ds(pl.multiple_of(w * 16, 16), 16)], cntv)
    cnt = cntv[...][0]
    nch = (cnt + (CHE - 1)) // CHE

    pltpu.sync_copy(wqk_hbm, wqkb)
    pltpu.sync_copy(wv_hbm, wvb)
    pltpu.sync_copy(bias_hbm, biasb)

    neg = jnp.full((16,), -1e30, jnp.float32)
    def minit(i, _):
        mb[pl.ds(i * 16, 16)] = neg
        return 0
    lax.fori_loop(0, MZ // 16, minit, 0)
    _zero16(zb, MZ)

    def unpack(j, _):
        rec = recb[pl.ds(j * 16, 16)]
        srcv = rec & 0xFFFF
        etv = lax.shift_right_logical(rec, 16) & 0xF
        dlv = lax.shift_right_logical(rec, 20) & 0x7FF
        kidx[pl.ds(j * 16, 16)] = srcv
        etb[pl.ds(j * 16, 16)] = etv
        dlb[pl.ds(j * 16, 16)] = dlv
        qidx[pl.ds(j * 16, 16)] = dlv + (w * BS)
        return 0

    def p1(ch, _):
        roff = pl.multiple_of(w * CAPB + ch * CHE, CHE)
        pltpu.sync_copy(recs_hbm.at[pl.ds(roff, CHE)], recb)
        lax.fori_loop(0, CHE // 16, unpack, 0)
        cpq = pltpu.async_copy(qt_hbm.at[qidx], qrows, semq)
        cpk = pltpu.async_copy(kt_hbm.at[kidx], krows, semk)
        cpq.wait()
        cpk.wait()

        def edge(e, _):
            etv = plsc.load_gather(etb, [jnp.full((16,), 0, jnp.int32) + e])
            wbase = etv * 64
            sv = jnp.zeros((16,), jnp.float32)
            for h in range(NH):
                qv = qrows[e, pl.ds(h * DK, DK)]
                kv = krows[e, pl.ds(h * DK, DK)]
                wv_ = plsc.load_gather(wqkb, [wbase + (h * DK) + iota])
                sh = jnp.sum(qv * kv * wv_)
                sv = sv + jnp.where(iota == h, sh, 0.0)
            bv = plsc.load_gather(biasb, [etv * 4 + h4])
            sv = sv + bv
            plsc.store_scatter(sbuf, [e * 4 + iota], sv, mask=iota < 4)
            return 0
        lax.fori_loop(0, CHE, edge, 0)

        def grp(g, _):
            dlq = plsc.load_gather(dlb, [g * 4 + e4])
            idxv = dlq * 4 + h4
            sv = sbuf[pl.ds(g * 16, 16)]
            cur = plsc.load_gather(mb, [idxv])
            plsc.store_scatter(mb, [idxv], jnp.maximum(cur, sv))
            def retry(_i, _c):
                chk = plsc.load_gather(mb, [idxv])
                need = chk < sv
                @pl.when(plsc.all_reduce_population_count(need)[0] > 0)
                def _():
                    cur2 = plsc.load_gather(mb, [idxv])
                    plsc.store_scatter(mb, [idxv], jnp.maximum(cur2, sv),
                                      mask=need)
                return 0
            lax.fori_loop(0, 3, retry, 0)
            return 0
        lax.fori_loop(0, CHE // 4, grp, 0)
        soff = pl.multiple_of(w * (CAPB * 4) + ch * (CHE * 4), CHE * 4)
        pltpu.sync_copy(sbuf, sc_hbm.at[pl.ds(soff, CHE * 4)])
        return 0
    lax.fori_loop(0, nch, p1, 0)

    nch2 = (cnt + 2047) // 2048
    def p2(ch, _):
        roff = pl.multiple_of(w * CAPB + ch * 2048, 2048)
        pltpu.sync_copy(recs_hbm.at[pl.ds(roff, 2048)], recb2)
        soff = pl.multiple_of(w * (CAPB * 4) + ch * 8192, 8192)
        pltpu.sync_copy(sc_hbm.at[pl.ds(soff, 8192)], sbuf2)
        def grp(g, _):
            rr = plsc.load_gather(recb2, [g * 4 + e4])
            dlq = lax.shift_right_logical(rr, 20) & 0x7FF
            idxv = dlq * 4 + h4
            sv = sbuf2[pl.ds(g * 16, 16)]
            mg = plsc.load_gather(mb, [idxv])
            es = jnp.exp(sv - mg)
            plsc.addupdate_scatter(zb, [idxv], es)
            return 0
        lax.fori_loop(0, 512, grp, 0)
        return 0
    lax.fori_loop(0, nch2, p2, 0)

    zv16 = jnp.zeros((16,), jnp.float32)
    for hh in range(2):
        def zagg(i, _):
            def zc(j, _2):
                aggv[i, pl.ds(j * 16, 16)] = zv16
                return 0
            lax.fori_loop(0, 2, zc, 0)
            return 0
        lax.fori_loop(0, SLABR - 4, zagg, 0)

        def p3(ch, _):
            roff = pl.multiple_of(w * CAPB + ch * CHE, CHE)
            pltpu.sync_copy(recs_hbm.at[pl.ds(roff, CHE)], recb)
            lax.fori_loop(0, CHE // 16, unpack, 0)
            def vb(j, _):
                vidxb[pl.ds(j * 16, 16)] = kidx[pl.ds(j * 16, 16)] * 2 + hh
                return 0
            lax.fori_loop(0, CHE // 16, vb, 0)
            soff = pl.multiple_of(w * (CAPB * 4) + ch * (CHE * 4), CHE * 4)
            pltpu.sync_copy(sc_hbm.at[pl.ds(soff, CHE * 4)], sbuf)
            cpv = pltpu.async_copy(vt_hbm.at[vidxb], krows2, semk)

            def grp(g, _):
                dlq = plsc.load_gather(dlb, [g * 4 + e4])
                idxv = dlq * 4 + h4
                sv = sbuf[pl.ds(g * 16, 16)]
                mg = plsc.load_gather(mb, [idxv])
                zg = plsc.load_gather(zb, [idxv])
                av = jnp.exp(sv - mg) / (zg + 1e-9)
                abuf[pl.ds(g * 16, 16)] = av
                return 0
            lax.fori_loop(0, CHE // 4, grp, 0)
            cpv.wait()

            def medge(e, _):
                etv = plsc.load_gather(etb, [jnp.full((16,), 0, jnp.int32) + e])
                wbase = etv * 64
                dl_s = plsc.load_gather(dlb, [jnp.full((16,), 0, jnp.int32) + e])[0]
                for h2 in range(2):
                    h = hh * 2 + h2
                    vv = krows2[e, pl.ds(h2 * DK, DK)]
                    wv_ = plsc.load_gather(wvb, [wbase + (h * DK) + iota])
                    av = plsc.load_gather(
                        abuf, [jnp.full((16,), 0, jnp.int32) + (e * 4 + h)])
                    aggv[dl_s, pl.ds(h2 * DK, DK)] += vv * wv_ * av
                return 0
            lax.fori_loop(0, CHE, medge, 0)
            return 0
        lax.fori_loop(0, nch, p3, 0)

        pltpu.sync_copy(aggv.at[pl.ds(0, BS)],
                        agg_hbm.at[hh].at[pl.ds(w * BS, BS)])


def _layer_edge(recs, cnts, qt, kt, vt, wqk, wv, bias):
    return pl.kernel(
        _layer_body,
        out_type=(jax.ShapeDtypeStruct((2, NPAD, HID // 2), jnp.float32),
                  jax.ShapeDtypeStruct((NW * CAPB * 4,), jnp.float32)),
        mesh=_sc_mesh(),
        compiler_params=_CP_SC,
        scratch_types=[
            pltpu.VMEM((CHE,), jnp.int32),
            pltpu.VMEM((2048,), jnp.int32),
            pltpu.VMEM((8192,), jnp.float32),
            pltpu.VMEM((CHE,), jnp.int32),
            pltpu.VMEM((CHE,), jnp.int32),
            pltpu.VMEM((CHE,), jnp.int32),
            pltpu.VMEM((CHE,), jnp.int32),
            pltpu.VMEM((CHE,), jnp.int32),
            pltpu.VMEM((CHE, HID), jnp.float32),
            pltpu.VMEM((CHE, HID), jnp.float32),
            pltpu.VMEM((CHE, HID // 2), jnp.float32),
            pltpu.VMEM((CHE * 4,), jnp.float32),
            pltpu.VMEM((CHE * 4,), jnp.float32),
            pltpu.VMEM((MZ,), jnp.float32),
            pltpu.VMEM((MZ,), jnp.float32),
            pltpu.VMEM((NREL * HID,), jnp.float32),
            pltpu.VMEM((NREL * HID,), jnp.float32),
            pltpu.VMEM((NREL * NH,), jnp.float32),
            pltpu.VMEM((SLABR - 4, HID // 2), jnp.float32),
            pltpu.VMEM((16,), jnp.int32),
            pltpu.SemaphoreType.DMA,
            pltpu.SemaphoreType.DMA,
        ],
    )(recs, cnts, qt, kt, vt, wqk, wv, bias)


def kernel(x, params, node_type, edge_index, edge_type):
    p = params
    xp = jnp.zeros((NPAD, DIN), jnp.float32).at[:NN].set(x)
    nt2 = jnp.zeros((NPAD, 1), jnp.int32).at[:NN, 0].set(node_type)

    src = edge_index[0].astype(jnp.int32)
    dst = edge_index[1].astype(jnp.int32)
    et = edge_type.astype(jnp.int32)
    pad = EP - EE
    srcp = jnp.concatenate([src, jnp.zeros((pad,), jnp.int32)])
    dstp = jnp.concatenate([dst, jnp.full((pad,), 1 << 26, jnp.int32)])
    etp = jnp.concatenate([et, jnp.zeros((pad,), jnp.int32)])

    pk2 = _pack_records(srcp.reshape(EP // 128, 128),
                        dstp.reshape(EP // 128, 128),
                        etp.reshape(EP // 128, 128))
    recs, cnts = _bin_edges(dstp, pk2.reshape(EP))

    h = _encoder(xp, nt2, p)

    alphas = jax.nn.sigmoid(p["skip"])
    out = None
    for l in range(NLAYER):
        qt, kt, vt = _qkv(h, nt2, p["Wq"][l], p["bq"][l], p["Wk"][l],
                          p["bk"][l], p["Wv"][l], p["bv"][l])
        wqk = (0.25 * p["rel_q"][l] * p["rel_k"][l]
               * p["sign_k"][l][None]).reshape(-1)
        wv = (p["rel_v"][l] * p["sign_v"][l][None]).reshape(-1)
        bias = p["rel_bias"][l].reshape(-1)
        vt2 = vt.reshape(2 * NPAD, HID // 2)
        agg2, _sc = _layer_edge(recs, cnts, qt, kt, vt2, wqk, wv, bias)
        agg = agg2.transpose(1, 0, 2).reshape(NPAD, HID)
        al8 = jnp.zeros((8, 128), jnp.float32).at[0, :NTY].set(alphas[l])
        h = _lnskip(agg, h, nt2, al8, p["ln_w"][l], p["ln_b"][l])

    out = _final_proj(h, p["Wmu"], p["bmu"])
    return out[:NN]


# speedup vs baseline: 1.1876x; 1.0132x over previous
import functools

import jax
import jax.numpy as jnp
from jax import lax
from jax.experimental import pallas as pl
from jax.experimental.pallas import tpu as pltpu
from jax.experimental.pallas import tpu_sc as plsc

NN = 50000
EE = 800000
DIN = 128
HID = 64
NH = 4
DK = 16
NTY = 3
NREL = 16
NLAYER = 2

NPAD = 50176
NBLK = 98
BS = 1563
NW = 32
SLABR = 1568
MZ = 6272
EP = 802816
CAPB = 806912
CHE = 256
SENT_REC = 1563 << 20


def _pack_body(src_ref, dst_ref, et_ref, pk_ref):
    d = dst_ref[...]
    b = jnp.zeros_like(d)
    for w in range(1, NW):
        b = b + jnp.where(d >= w * BS, 1, 0)
    dl = d - b * BS
    pk_ref[...] = src_ref[...] | (et_ref[...] << 16) | (dl << 20)


def _pack_records(src2, dst2, et2):
    return pl.pallas_call(
        _pack_body,
        grid=(1,),
        in_specs=[pl.BlockSpec((EP // 128, 128), lambda i: (0, 0))] * 3,
        out_specs=pl.BlockSpec((EP // 128, 128), lambda i: (0, 0)),
        out_shape=jax.ShapeDtypeStruct((EP // 128, 128), jnp.int32),
    )(src2, dst2, et2)


def _enc_body(x_ref, nt_ref, w1_ref, b1_ref, w2_ref, b2_ref, o_ref):
    x = x_ref[...]
    nt = nt_ref[...]
    h = jnp.zeros((512, HID), jnp.float32)
    for t in range(NTY):
        ht = jax.nn.relu(x @ w1_ref[t] + b1_ref[t][None, :])
        ht = ht @ w2_ref[t] + b2_ref[t][None, :]
        h = jnp.where(nt == t, ht, h)
    o_ref[...] = h


def _encoder(xp, nt2, p):
    return pl.pallas_call(
        _enc_body,
        grid=(NBLK,),
        in_specs=[
            pl.BlockSpec((512, DIN), lambda i: (i, 0)),
            pl.BlockSpec((512, 1), lambda i: (i, 0)),
            pl.BlockSpec((NTY, DIN, DIN), lambda i: (0, 0, 0)),
            pl.BlockSpec((NTY, DIN), lambda i: (0, 0)),
            pl.BlockSpec((NTY, DIN, HID), lambda i: (0, 0, 0)),
            pl.BlockSpec((NTY, HID), lambda i: (0, 0)),
        ],
        out_specs=pl.BlockSpec((512, HID), lambda i: (i, 0)),
        out_shape=jax.ShapeDtypeStruct((NPAD, HID), jnp.float32),
    )(xp, nt2, p["enc_W1"], p["enc_b1"], p["enc_W2"], p["enc_b2"])


def _qkv_body(h_ref, nt_ref, wq_ref, bq_ref, wk_ref, bk_ref, wv_ref, bv_ref,
              q_ref, k_ref, v_ref):
    h = h_ref[...]
    nt = nt_ref[...]
    q = jnp.zeros((512, HID), jnp.float32)
    k = jnp.zeros((512, HID), jnp.float32)
    v = jnp.zeros((512, HID), jnp.float32)
    for t in range(NTY):
        m = nt == t
        q = jnp.where(m, h @ wq_ref[t] + bq_ref[t][None, :], q)
        k = jnp.where(m, h @ wk_ref[t] + bk_ref[t][None, :], k)
        v = jnp.where(m, h @ wv_ref[t] + bv_ref[t][None, :], v)
    q_ref[...] = q
    k_ref[...] = k
    v_ref[...] = v


def _qkv(h, nt2, wq, bq, wk, bk, wv, bv):
    spec = pl.BlockSpec((512, HID), lambda i: (i, 0))
    wspec = pl.BlockSpec((NTY, HID, HID), lambda i: (0, 0, 0))
    bspec = pl.BlockSpec((NTY, HID), lambda i: (0, 0))
    sh = jax.ShapeDtypeStruct((NPAD, HID), jnp.float32)
    return pl.pallas_call(
        _qkv_body,
        grid=(NBLK,),
        in_specs=[spec, pl.BlockSpec((512, 1), lambda i: (i, 0)),
                  wspec, bspec, wspec, bspec, wspec, bspec],
        out_specs=[spec, spec, spec],
        out_shape=[sh, sh, sh],
    )(h, nt2, wq, bq, wk, bk, wv, bv)


def _ln_body(agg_ref, h_ref, nt_ref, al_ref, lw_ref, lb_ref, o_ref):
    agg = agg_ref[...]
    h = h_ref[...]
    nt = nt_ref[...]
    out = jnp.zeros((512, HID), jnp.float32)
    for t in range(NTY):
        alpha = al_ref[0, t]
        y = alpha * agg + (1.0 - alpha) * h
        mu = y.mean(-1, keepdims=True)
        var = ((y - mu) ** 2).mean(-1, keepdims=True)
        y = (y - mu) / jnp.sqrt(var + 1e-5) * lw_ref[t][None, :] + lb_ref[t][None, :]
        out = jnp.where(nt == t, y, out)
    o_ref[...] = out


def _lnskip(agg, h, nt2, alphas8, lw, lb):
    spec = pl.BlockSpec((512, HID), lambda i: (i, 0))
    return pl.pallas_call(
        _ln_body,
        grid=(NBLK,),
        in_specs=[spec, spec, pl.BlockSpec((512, 1), lambda i: (i, 0)),
                  pl.BlockSpec((8, 128), lambda i: (0, 0)),
                  pl.BlockSpec((NTY, HID), lambda i: (0, 0)),
                  pl.BlockSpec((NTY, HID), lambda i: (0, 0))],
        out_specs=spec,
        out_shape=jax.ShapeDtypeStruct((NPAD, HID), jnp.float32),
    )(agg, h, nt2, alphas8, lw, lb)


def _proj_body(h_ref, w_ref, b_ref, o_ref):
    o_ref[...] = h_ref[...] @ w_ref[...] + b_ref[...]


def _final_proj(h, W, b):
    return pl.pallas_call(
        _proj_body,
        grid=(NBLK,),
        in_specs=[
            pl.BlockSpec((512, HID), lambda i: (i, 0)),
            pl.BlockSpec((HID, HID), lambda i: (0, 0)),
            pl.BlockSpec((1, HID), lambda i: (0, 0)),
        ],
        out_specs=pl.BlockSpec((512, HID), lambda i: (i, 0)),
        out_shape=jax.ShapeDtypeStruct((NPAD, HID), jnp.float32),
    )(h, W, b.reshape(1, HID))


_CP_SC = pltpu.CompilerParams(use_tc_tiling_on_sc=False, needs_layout_passes=False)


def _sc_mesh():
    return plsc.VectorSubcoreMesh(core_axis_name="c", subcore_axis_name="s")


def _zero16(ref, n):
    zv = jnp.zeros((16,), jnp.float32)
    def b(i, _):
        ref[pl.ds(i * 16, 16)] = zv
        return 0
    lax.fori_loop(0, n // 16, b, 0)


def _bin_body(dst_hbm, pk_hbm, recs_hbm, cnt_hbm,
              dstb, pkb, outb, sentb, tmpb, cntv):
    c = lax.axis_index("c")
    s = lax.axis_index("s")
    w = s * 2 + c
    lo = w * BS
    hi = lo + BS
    sent = jnp.full((16,), SENT_REC, jnp.int32)
    def fill_sent(ref, n16):
        def b(i, _):
            ref[pl.ds(i * 16, 16)] = sent
            return 0
        lax.fori_loop(0, n16, b, 0)
    fill_sent(sentb, 128)
    fill_sent(outb, 256)

    def chunk(j, carry):
        f, wpos = carry
        pltpu.sync_copy(dst_hbm.at[pl.ds(j * 8192, 8192)], dstb)
        pltpu.sync_copy(pk_hbm.at[pl.ds(j * 8192, 8192)], pkb)
        def vec(i, f):
            d = dstb[pl.ds(i * 16, 16)]
            pk = pkb[pl.ds(i * 16, 16)]
            m = (d >= lo) & (d < hi)
            mi = jnp.where(m, 1, 0)
            pos = f + plsc.cumsum(mi) - 1
            plsc.store_scatter(outb, [pos], pk, mask=m)
            return f + plsc.all_reduce_population_count(m)[0]

        def flush(carry):
            f, wpos = carry
            @pl.when(f >= 2048)
            def _():
                off = pl.multiple_of(w * CAPB + wpos, 2048)
                pltpu.sync_copy(outb.at[pl.ds(0, 2048)],
                                recs_hbm.at[pl.ds(off, 2048)])
                def shift(i, _):
                    outb[pl.ds(i * 16, 16)] = outb[pl.ds(2048 + i * 16, 16)]
                    outb[pl.ds(2048 + i * 16, 16)] = sent
                    return 0
                lax.fori_loop(0, 128, shift, 0)
            return (jnp.where(f >= 2048, f - 2048, f),
                    jnp.where(f >= 2048, wpos + 2048, wpos))

        def sub(k2, carry):
            f, wpos = carry
            f = lax.fori_loop(k2 * 128, (k2 + 1) * 128, vec, f)
            f, wpos = flush((f, wpos))
            f, wpos = flush((f, wpos))
            return (f, wpos)
        return lax.fori_loop(0, 4, sub, (f, wpos))

    f, wpos = lax.fori_loop(0, EP // 8192, chunk, (jnp.int32(0), jnp.int32(0)))
    off = pl.multiple_of(w * CAPB + wpos, 2048)
    pltpu.sync_copy(outb.at[pl.ds(0, 2048)], recs_hbm.at[pl.ds(off, 2048)])
    off2 = pl.multiple_of(w * CAPB + wpos + 2048, 2048)
    pltpu.sync_copy(sentb, recs_hbm.at[pl.ds(off2, 2048)])
    cntv[...] = jnp.full((16,), wpos + f, jnp.int32)
    pltpu.sync_copy(cntv, cnt_hbm.at[pl.ds(pl.multiple_of(w * 16, 16), 16)])


def _bin_edges(dst_flat, pk_flat):
    return pl.kernel(
        _bin_body,
        out_type=(jax.ShapeDtypeStruct((NW * CAPB,), jnp.int32),
                  jax.ShapeDtypeStruct((NW * 16,), jnp.int32)),
        mesh=_sc_mesh(),
        compiler_params=_CP_SC,
        scratch_types=[
            pltpu.VMEM((8192,), jnp.int32),
            pltpu.VMEM((8192,), jnp.int32),
            pltpu.VMEM((4096,), jnp.int32),
            pltpu.VMEM((2048,), jnp.int32),
            pltpu.VMEM((2048,), jnp.int32),
            pltpu.VMEM((16,), jnp.int32),
        ],
    )(dst_flat, pk_flat)


def _layer_body(recs_hbm, cnt_hbm, qt_hbm, kt_hbm, vt_hbm,
                wqk_hbm, wv_hbm, bias_hbm,
                agg_hbm, sc_hbm,
                recb, recb2, sbuf2, qidx, kidx, dlb, etb, vidxb,
                qrows, krows, krows2, sbuf, abuf,
                mb, zb, wqkb, wvb, biasb, aggv, cntv,
                semq, semk):
    c = lax.axis_index("c")
    s = lax.axis_index("s")
    w = s * 2 + c
    iota = lax.iota(jnp.int32, 16)
    e4 = lax.shift_right_logical(iota, 2)
    h4 = iota & 3

    pltpu.sync_copy(cnt_hbm.at[pl.ds(pl.multiple_of(w * 16, 16), 16)], cntv)
    cnt = cntv[...][0]
    nch = (cnt + (CHE - 1)) // CHE

    pltpu.sync_copy(wqk_hbm, wqkb)
    pltpu.sync_copy(wv_hbm, wvb)
    pltpu.sync_copy(bias_hbm, biasb)

    neg = jnp.full((16,), -1e30, jnp.float32)
    def minit(i, _):
        mb[pl.ds(i * 16, 16)] = neg
        return 0
    lax.fori_loop(0, MZ // 16, minit, 0)
    _zero16(zb, MZ)

    def unpack(j, _):
        rec = recb[pl.ds(j * 16, 16)]
        srcv = rec & 0xFFFF
        etv = lax.shift_right_logical(rec, 16) & 0xF
        dlv = lax.shift_right_logical(rec, 20) & 0x7FF
        kidx[pl.ds(j * 16, 16)] = srcv
        etb[pl.ds(j * 16, 16)] = etv
        dlb[pl.ds(j * 16, 16)] = dlv
        qidx[pl.ds(j * 16, 16)] = dlv + (w * BS)
        return 0

    def p1(ch, _):
        roff = pl.multiple_of(w * CAPB + ch * CHE, CHE)
        pltpu.sync_copy(recs_hbm.at[pl.ds(roff, CHE)], recb)
        lax.fori_loop(0, CHE // 16, unpack, 0)
        cpq = pltpu.async_copy(qt_hbm.at[qidx], qrows, semq)
        cpk = pltpu.async_copy(kt_hbm.at[kidx], krows, semk)
        cpq.wait()
        cpk.wait()

        def edge2(ep, _):
            for u in range(2):
                e = ep * 2 + u
                etv = plsc.load_gather(etb, [jnp.full((16,), 0, jnp.int32) + e])
                wbase = etv * 64
                sv = jnp.zeros((16,), jnp.float32)
                for h in range(NH):
                    qv = qrows[e, pl.ds(h * DK, DK)]
                    kv = krows[e, pl.ds(h * DK, DK)]
                    wv_ = plsc.load_gather(wqkb, [wbase + (h * DK) + iota])
                    sh = jnp.sum(qv * kv * wv_)
                    sv = sv + jnp.where(iota == h, sh, 0.0)
                bv = plsc.load_gather(biasb, [etv * 4 + h4])
                sv = sv + bv
                plsc.store_scatter(sbuf, [e * 4 + iota], sv, mask=iota < 4)
            return 0
        lax.fori_loop(0, CHE // 2, edge2, 0)

        def grp(g, _):
            dlq = plsc.load_gather(dlb, [g * 4 + e4])
            idxv = dlq * 4 + h4
            sv = sbuf[pl.ds(g * 16, 16)]
            cur = plsc.load_gather(mb, [idxv])
            plsc.store_scatter(mb, [idxv], jnp.maximum(cur, sv))
            def retry(_i, _c):
                chk = plsc.load_gather(mb, [idxv])
                need = chk < sv
                @pl.when(plsc.all_reduce_population_count(need)[0] > 0)
                def _():
                    cur2 = plsc.load_gather(mb, [idxv])
                    plsc.store_scatter(mb, [idxv], jnp.maximum(cur2, sv),
                                      mask=need)
                return 0
            lax.fori_loop(0, 3, retry, 0)
            return 0
        lax.fori_loop(0, CHE // 4, grp, 0)
        soff = pl.multiple_of(w * (CAPB * 4) + ch * (CHE * 4), CHE * 4)
        pltpu.sync_copy(sbuf, sc_hbm.at[pl.ds(soff, CHE * 4)])
        return 0
    lax.fori_loop(0, nch, p1, 0)

    nch2 = (cnt + 2047) // 2048
    def p2(ch, _):
        roff = pl.multiple_of(w * CAPB + ch * 2048, 2048)
        pltpu.sync_copy(recs_hbm.at[pl.ds(roff, 2048)], recb2)
        soff = pl.multiple_of(w * (CAPB * 4) + ch * 8192, 8192)
        pltpu.sync_copy(sc_hbm.at[pl.ds(soff, 8192)], sbuf2)
        def grp(g, _):
            rr = plsc.load_gather(recb2, [g * 4 + e4])
            dlq = lax.shift_right_logical(rr, 20) & 0x7FF
            idxv = dlq * 4 + h4
            sv = sbuf2[pl.ds(g * 16, 16)]
            mg = plsc.load_gather(mb, [idxv])
            es = jnp.exp(sv - mg)
            plsc.addupdate_scatter(zb, [idxv], es)
            return 0
        lax.fori_loop(0, 512, grp, 0)
        return 0
    lax.fori_loop(0, nch2, p2, 0)

    zv16 = jnp.zeros((16,), jnp.float32)
    for hh in range(2):
        def zagg(i, _):
            def zc(j, _2):
                aggv[i, pl.ds(j * 16, 16)] = zv16
                return 0
            lax.fori_loop(0, 2, zc, 0)
            return 0
        lax.fori_loop(0, SLABR - 4, zagg, 0)

        def p3(ch, _):
            roff = pl.multiple_of(w * CAPB + ch * CHE, CHE)
            pltpu.sync_copy(recs_hbm.at[pl.ds(roff, CHE)], recb)
            lax.fori_loop(0, CHE // 16, unpack, 0)
            def vb(j, _):
                vidxb[pl.ds(j * 16, 16)] = kidx[pl.ds(j * 16, 16)] * 2 + hh
                return 0
            lax.fori_loop(0, CHE // 16, vb, 0)
            soff = pl.multiple_of(w * (CAPB * 4) + ch * (CHE * 4), CHE * 4)
            pltpu.sync_copy(sc_hbm.at[pl.ds(soff, CHE * 4)], sbuf)
            cpv = pltpu.async_copy(vt_hbm.at[vidxb], krows2, semk)

            def grp(g, _):
                dlq = plsc.load_gather(dlb, [g * 4 + e4])
                idxv = dlq * 4 + h4
                sv = sbuf[pl.ds(g * 16, 16)]
                mg = plsc.load_gather(mb, [idxv])
                zg = plsc.load_gather(zb, [idxv])
                av = jnp.exp(sv - mg) / (zg + 1e-9)
                abuf[pl.ds(g * 16, 16)] = av
                return 0
            lax.fori_loop(0, CHE // 4, grp, 0)
            cpv.wait()

            def medge2(ep, _):
                for u in range(2):
                    e = ep * 2 + u
                    etv = plsc.load_gather(etb, [jnp.full((16,), 0, jnp.int32) + e])
                    wbase = etv * 64
                    dl_s = plsc.load_gather(dlb, [jnp.full((16,), 0, jnp.int32) + e])[0]
                    for h2 in range(2):
                        h = hh * 2 + h2
                        vv = krows2[e, pl.ds(h2 * DK, DK)]
                        wv_ = plsc.load_gather(wvb, [wbase + (h * DK) + iota])
                        av = plsc.load_gather(
                            abuf, [jnp.full((16,), 0, jnp.int32) + (e * 4 + h)])
                        aggv[dl_s, pl.ds(h2 * DK, DK)] += vv * wv_ * av
                return 0
            lax.fori_loop(0, CHE // 2, medge2, 0)
            return 0
        lax.fori_loop(0, nch, p3, 0)

        pltpu.sync_copy(aggv.at[pl.ds(0, BS)],
                        agg_hbm.at[hh].at[pl.ds(w * BS, BS)])


def _layer_edge(recs, cnts, qt, kt, vt, wqk, wv, bias):
    return pl.kernel(
        _layer_body,
        out_type=(jax.ShapeDtypeStruct((2, NPAD, HID // 2), jnp.float32),
                  jax.ShapeDtypeStruct((NW * CAPB * 4,), jnp.float32)),
        mesh=_sc_mesh(),
        compiler_params=_CP_SC,
        scratch_types=[
            pltpu.VMEM((CHE,), jnp.int32),
            pltpu.VMEM((2048,), jnp.int32),
            pltpu.VMEM((8192,), jnp.float32),
            pltpu.VMEM((CHE,), jnp.int32),
            pltpu.VMEM((CHE,), jnp.int32),
            pltpu.VMEM((CHE,), jnp.int32),
            pltpu.VMEM((CHE,), jnp.int32),
            pltpu.VMEM((CHE,), jnp.int32),
            pltpu.VMEM((CHE, HID), jnp.float32),
            pltpu.VMEM((CHE, HID), jnp.float32),
            pltpu.VMEM((CHE, HID // 2), jnp.float32),
            pltpu.VMEM((CHE * 4,), jnp.float32),
            pltpu.VMEM((CHE * 4,), jnp.float32),
            pltpu.VMEM((MZ,), jnp.float32),
            pltpu.VMEM((MZ,), jnp.float32),
            pltpu.VMEM((NREL * HID,), jnp.float32),
            pltpu.VMEM((NREL * HID,), jnp.float32),
            pltpu.VMEM((NREL * NH,), jnp.float32),
            pltpu.VMEM((SLABR - 4, HID // 2), jnp.float32),
            pltpu.VMEM((16,), jnp.int32),
            pltpu.SemaphoreType.DMA,
            pltpu.SemaphoreType.DMA,
        ],
    )(recs, cnts, qt, kt, vt, wqk, wv, bias)


def kernel(x, params, node_type, edge_index, edge_type):
    p = params
    xp = jnp.zeros((NPAD, DIN), jnp.float32).at[:NN].set(x)
    nt2 = jnp.zeros((NPAD, 1), jnp.int32).at[:NN, 0].set(node_type)

    src = edge_index[0].astype(jnp.int32)
    dst = edge_index[1].astype(jnp.int32)
    et = edge_type.astype(jnp.int32)
    pad = EP - EE
    srcp = jnp.concatenate([src, jnp.zeros((pad,), jnp.int32)])
    dstp = jnp.concatenate([dst, jnp.full((pad,), 1 << 26, jnp.int32)])
    etp = jnp.concatenate([et, jnp.zeros((pad,), jnp.int32)])

    pk2 = _pack_records(srcp.reshape(EP // 128, 128),
                        dstp.reshape(EP // 128, 128),
                        etp.reshape(EP // 128, 128))
    recs, cnts = _bin_edges(dstp, pk2.reshape(EP))

    h = _encoder(xp, nt2, p)

    alphas = jax.nn.sigmoid(p["skip"])
    out = None
    for l in range(NLAYER):
        qt, kt, vt = _qkv(h, nt2, p["Wq"][l], p["bq"][l], p["Wk"][l],
                          p["bk"][l], p["Wv"][l], p["bv"][l])
        wqk = (0.25 * p["rel_q"][l] * p["rel_k"][l]
               * p["sign_k"][l][None]).reshape(-1)
        wv = (p["rel_v"][l] * p["sign_v"][l][None]).reshape(-1)
        bias = p["rel_bias"][l].reshape(-1)
        vt2 = vt.reshape(2 * NPAD, HID // 2)
        agg2, _sc = _layer_edge(recs, cnts, qt, kt, vt2, wqk, wv, bias)
        agg = agg2.transpose(1, 0, 2).reshape(NPAD, HID)
        al8 = jnp.zeros((8, 128), jnp.float32).at[0, :NTY].set(alphas[l])
        h = _lnskip(agg, h, nt2, al8, p["ln_w"][l], p["ln_b"][l])

    out = _final_proj(h, p["Wmu"], p["bmu"])
    return out[:NN]
